# baseline (speedup 1.0000x reference)
import functools

import jax
import jax.numpy as jnp
from jax import lax
from jax.experimental import pallas as pl
from jax.experimental.pallas import tpu as pltpu
from jax.experimental.pallas import tpu_sc as plsc

SCALE = 8.0
NC = 2
NS = 16
NW = NC * NS
G = 128
CHUNK = 1024
CG = CHUNK // G


@functools.lru_cache(maxsize=None)
def _make_emb(B, V, D):
    bpw = B // NW
    nchunk = bpw // CHUNK
    mesh = plsc.VectorSubcoreMesh(core_axis_name="c", subcore_axis_name="s")

    @functools.partial(
        pl.kernel,
        mesh=mesh,
        compiler_params=pltpu.CompilerParams(use_tc_tiling_on_sc=False),
        out_type=jax.ShapeDtypeStruct((B, D), jnp.float32),
        scratch_types=[
            pltpu.VMEM((CG, G), jnp.int32),
            pltpu.VMEM((CHUNK, D), jnp.float32),
            pltpu.SemaphoreType.DMA,
        ],
    )
    def emb(idx_hbm, table_hbm, out_hbm, idx_v, rows_v, sem):
        wid = lax.axis_index("s") * NC + lax.axis_index("c")
        row0 = wid * bpw
        grp0 = row0 // G

        def chunk_body(c, carry):
            goff = pl.multiple_of(grp0 + c * CG, 8)
            roff = pl.multiple_of(row0 + c * CHUNK, CHUNK)
            pltpu.sync_copy(idx_hbm.at[pl.ds(goff, CG)], idx_v)
            copies = []
            for g in range(CG):
                copies.append(pltpu.async_copy(
                    table_hbm.at[idx_v.at[g]],
                    rows_v.at[pl.ds(g * G, G)], sem))
            for cp in copies:
                cp.wait()

            def scale_body(i, c2):
                r0 = i * 8
                for r in range(8):
                    for j in range(D // 16):
                        s = (r0 + r, pl.ds(j * 16, 16))
                        rows_v[s] = rows_v[s] * SCALE
                return c2

            lax.fori_loop(0, CHUNK // 8, scale_body, 0)
            pltpu.sync_copy(rows_v, out_hbm.at[pl.ds(roff, CHUNK)])
            return carry

        lax.fori_loop(0, nchunk, chunk_body, 0)

    return emb


def kernel(X, table):
    R, S = X.shape
    V, D = table.shape
    B = R * S
    idx = X.reshape(B // G, G).astype(jnp.int32)
    out = _make_emb(B, V, D)(idx, table)
    return out.reshape(R, S, D)


# speedup vs baseline: 1.0315x; 1.0315x over previous
import functools

import jax
import jax.numpy as jnp
from jax import lax
from jax.experimental import pallas as pl
from jax.experimental.pallas import tpu as pltpu
from jax.experimental.pallas import tpu_sc as plsc

SCALE = 8.0
NC = 2
NS = 16
NW = NC * NS
G = 128
CHUNK = 512
CG = CHUNK // G
NBUF = 3


@functools.lru_cache(maxsize=None)
def _make_emb(B, V, D):
    bpw = B // NW
    nchunk = bpw // CHUNK
    ngrp = bpw // G
    mesh = plsc.VectorSubcoreMesh(core_axis_name="c", subcore_axis_name="s")

    @functools.partial(
        pl.kernel,
        mesh=mesh,
        compiler_params=pltpu.CompilerParams(use_tc_tiling_on_sc=False),
        out_type=jax.ShapeDtypeStruct((B, D), jnp.float32),
        scratch_types=[
            pltpu.VMEM((ngrp, G), jnp.int32),
            pltpu.VMEM((NBUF, CHUNK, D), jnp.float32),
        ]
        + [pltpu.SemaphoreType.DMA for _ in range(2 * NBUF)],
    )
    def emb(idx_hbm, table_hbm, out_hbm, idx_v, rows_v, *sems):
        gsems = sems[:NBUF]
        ssems = sems[NBUF:]
        wid = lax.axis_index("s") * NC + lax.axis_index("c")
        row0 = wid * bpw
        grp0 = pl.multiple_of(wid * ngrp, 8)

        pltpu.sync_copy(idx_hbm.at[pl.ds(grp0, ngrp)], idx_v)

        def fire(c):
            b = c % NBUF
            return [
                pltpu.async_copy(table_hbm.at[idx_v.at[c * CG + g]],
                                 rows_v.at[b, pl.ds(g * G, G)], gsems[b])
                for g in range(CG)
            ]

        def scale(b):
            def scale_body(i, c2):
                r0 = i * 8
                for r in range(8):
                    for j in range(D // 16):
                        s = (b, r0 + r, pl.ds(j * 16, 16))
                        rows_v[s] = rows_v[s] * SCALE
                return c2
            lax.fori_loop(0, CHUNK // 8, scale_body, 0)

        ghandles = {}
        shandles = {}
        ghandles[0] = fire(0)
        if nchunk > 1:
            ghandles[1] = fire(1)
        for c in range(nchunk):
            b = c % NBUF
            n = c + 2
            if n < nchunk:
                if n >= NBUF:
                    shandles.pop(n - NBUF).wait()
                ghandles[n] = fire(n)
            for h in ghandles.pop(c):
                h.wait()
            scale(b)
            roff = row0 + c * CHUNK
            shandles[c] = pltpu.async_copy(
                rows_v.at[b], out_hbm.at[pl.ds(roff, CHUNK)], ssems[b])
        for c in sorted(shandles):
            shandles.pop(c).wait()

    return emb


def kernel(X, table):
    R, S = X.shape
    V, D = table.shape
    B = R * S
    idx = X.reshape(B // G, G).astype(jnp.int32)
    out = _make_emb(B, V, D)(idx, table)
    return out.reshape(R, S, D)
